# Initial kernel scaffold; baseline (speedup 1.0000x reference)
import jax
import jax.numpy as jnp
from jax.experimental import pallas as pl


def kernel(inputs, W_in, rows, cols, vals, bias):
    raise NotImplementedError("write your pallas kernel here")


# speedup vs baseline: 27.4630x; 27.4630x over previous
import functools

import jax
import jax.numpy as jnp
from jax import lax
from jax.experimental import pallas as pl
from jax.experimental.pallas import tpu as pltpu
from jax.experimental.pallas import tpu_sc as plsc

N = 4096
G = 4 * N
NNZ = 671088
DIN = 256
B = 32
T = 16

NC = 2
NS = 16
HB = B // NC
CH = 1024
NNZ_T = 41984
NCHUNK = NNZ_T // CH
NSTR = CH // 128
NSEG = N // NS
GSEG = G // NS
NNZ_PAD = NS * NNZ_T


def _xproj_body(x_ref, w_ref, b_ref, o_ref):
    o_ref[...] = (
        jnp.dot(x_ref[...], w_ref[...], preferred_element_type=jnp.float32)
        + b_ref[...][None, :]
    )


_xproj_call = pl.pallas_call(
    _xproj_body,
    grid=(G // 512,),
    in_specs=[
        pl.BlockSpec((B * T, DIN), lambda g: (0, 0)),
        pl.BlockSpec((DIN, 512), lambda g: (0, g)),
        pl.BlockSpec((512,), lambda g: (g,)),
    ],
    out_specs=pl.BlockSpec((B * T, 512), lambda g: (0, g)),
    out_shape=jax.ShapeDtypeStruct((B * T, G), jnp.float32),
)


def _splat(v, j):
    idx = jnp.full((16, 1), j, dtype=jnp.int32)
    dnums = lax.GatherDimensionNumbers(
        offset_dims=(), collapsed_slice_dims=(0,), start_index_map=(0,)
    )
    return lax.gather(
        v, idx, dnums, slice_sizes=(1,),
        mode=lax.GatherScatterMode.PROMISE_IN_BOUNDS,
    )


def _sigmoid(x):
    return 1.0 / (1.0 + jnp.exp(-x))


def _tanh(x):
    return 2.0 / (1.0 + jnp.exp(-2.0 * x)) - 1.0


def _sc_body(xp_hbm, rc_hbm, vals_hbm, out_hbm,
             rc_v, vbuf, rows2d, cols2d, gbuf,
             ibuf, fbuf, g2buf, obuf, cbuf, hbuf,
             h_sh, gates_sh, sem, sem2):
    c = lax.axis_index("c")
    s = lax.axis_index("s")
    tb = s * NNZ_T
    n0 = s * NSEG
    g0 = s * GSEG

    pltpu.sync_copy(rc_hbm.at[pl.ds(tb, NNZ_T)], rc_v)

    def _zero(r, carry):
        hbuf[r] = jnp.zeros((HB,), jnp.float32)
        cbuf[r] = jnp.zeros((HB,), jnp.float32)
        return carry

    lax.fori_loop(0, NSEG, _zero, 0)
    pltpu.sync_copy(hbuf, h_sh.at[pl.ds(n0, NSEG)])
    plsc.subcore_barrier()

    def _step(t, carry):
        pltpu.sync_copy(xp_hbm.at[c, t, pl.ds(g0, GSEG)], gbuf)
        pltpu.sync_copy(gbuf, gates_sh.at[pl.ds(g0, GSEG)])
        plsc.subcore_barrier()

        def _chunk(k, carry2):
            cb = k * CH
            pltpu.sync_copy(vals_hbm.at[pl.ds(tb + cb, CH)], vbuf)

            def _unpack(j, carry3):
                for gg in range(8):
                    v = rc_v[pl.ds(cb + j * 128 + gg * 16, 16)]
                    rows2d[j, pl.ds(gg * 16, 16)] = jnp.right_shift(v, 14)
                    cols2d[j, pl.ds(gg * 16, 16)] = jnp.bitwise_and(v, 0x3FFF)
                return carry3

            lax.fori_loop(0, NSTR, _unpack, 0)

            descs = [
                pltpu.async_copy(
                    h_sh.at[rows2d.at[j]], gbuf.at[pl.ds(j * 128, 128)], sem
                )
                for j in range(NSTR)
            ]
            for d in descs:
                d.wait()

            def _mult(q, carry3):
                vv = vbuf[pl.ds(q * 16, 16)]
                base = q * 16
                for j in range(16):
                    gbuf[base + j] = gbuf[base + j] * _splat(vv, j)
                return carry3

            lax.fori_loop(0, CH // 16, _mult, 0)

            descs2 = [
                pltpu.async_copy(
                    gbuf.at[pl.ds(j * 128, 128)],
                    gates_sh.at[cols2d.at[j]],
                    sem2,
                    add=True,
                )
                for j in range(NSTR)
            ]
            for d in descs2:
                d.wait()
            return carry2

        lax.fori_loop(0, NCHUNK, _chunk, 0)
        plsc.subcore_barrier()

        pltpu.sync_copy(gates_sh.at[pl.ds(n0, NSEG)], ibuf)
        pltpu.sync_copy(gates_sh.at[pl.ds(N + n0, NSEG)], fbuf)
        pltpu.sync_copy(gates_sh.at[pl.ds(2 * N + n0, NSEG)], g2buf)
        pltpu.sync_copy(gates_sh.at[pl.ds(3 * N + n0, NSEG)], obuf)

        def _ew(r, carry2):
            cn = _sigmoid(fbuf[r]) * cbuf[r] + _sigmoid(ibuf[r]) * _tanh(g2buf[r])
            cbuf[r] = cn
            hbuf[r] = _sigmoid(obuf[r]) * _tanh(cn)
            return carry2

        lax.fori_loop(0, NSEG, _ew, 0)
        pltpu.sync_copy(hbuf, h_sh.at[pl.ds(n0, NSEG)])
        pltpu.sync_copy(hbuf, out_hbm.at[c, t, pl.ds(n0, NSEG)])
        plsc.subcore_barrier()
        return carry

    lax.fori_loop(0, T, _step, 0)


_sc_call = pl.kernel(
    _sc_body,
    out_type=jax.ShapeDtypeStruct((NC, T, N, HB), jnp.float32),
    mesh=plsc.VectorSubcoreMesh(core_axis_name="c", subcore_axis_name="s"),
    scratch_types=[
        pltpu.VMEM((NNZ_T,), jnp.int32),
        pltpu.VMEM((CH,), jnp.float32),
        pltpu.VMEM((NSTR, 128), jnp.int32),
        pltpu.VMEM((NSTR, 128), jnp.int32),
        pltpu.VMEM((CH, HB), jnp.float32),
        pltpu.VMEM((NSEG, HB), jnp.float32),
        pltpu.VMEM((NSEG, HB), jnp.float32),
        pltpu.VMEM((NSEG, HB), jnp.float32),
        pltpu.VMEM((NSEG, HB), jnp.float32),
        pltpu.VMEM((NSEG, HB), jnp.float32),
        pltpu.VMEM((NSEG, HB), jnp.float32),
        pltpu.VMEM_SHARED((N, HB), jnp.float32),
        pltpu.VMEM_SHARED((G, HB), jnp.float32),
        pltpu.SemaphoreType.DMA,
        pltpu.SemaphoreType.DMA,
    ],
    compiler_params=pltpu.CompilerParams(use_tc_tiling_on_sc=False),
)


def kernel(inputs, W_in, rows, cols, vals, bias):
    xproj = _xproj_call(inputs.reshape(B * T, DIN), W_in, bias)
    xp4 = xproj.reshape(NC, HB, T, G).transpose(0, 2, 3, 1)

    rc = rows * jnp.int32(G) + cols
    npad = NNZ_PAD - NNZ
    pad_ar = jnp.arange(npad, dtype=jnp.int32)
    rc_p = jnp.concatenate([rc, (pad_ar % N) * jnp.int32(G) + pad_ar % G])
    vals_p = jnp.concatenate([vals, jnp.zeros((npad,), jnp.float32)])

    hs4 = _sc_call(xp4, rc_p, vals_p)
    return hs4.transpose(0, 3, 1, 2).reshape(B, T, N)


# speedup vs baseline: 49.7981x; 1.8133x over previous
import functools

import jax
import jax.numpy as jnp
from jax import lax
from jax.experimental import pallas as pl
from jax.experimental.pallas import tpu as pltpu
from jax.experimental.pallas import tpu_sc as plsc

N = 4096
G = 4 * N
NNZ = 671088
DIN = 256
B = 32
T = 16

NC = 2
NS = 16
HB = B // NC
CH = 512
NCHUNK = 84
NNZ_T = CH * NCHUNK
NSTR = CH // 128
NSEG = N // NS
GSEG = G // NS
NNZ_PAD = NS * NNZ_T


def _xproj_body(x_ref, w_ref, b_ref, o_ref):
    o_ref[...] = (
        jnp.dot(x_ref[...], w_ref[...], preferred_element_type=jnp.float32)
        + b_ref[...][None, :]
    )


_xproj_call = pl.pallas_call(
    _xproj_body,
    grid=(G // 512,),
    in_specs=[
        pl.BlockSpec((B * T, DIN), lambda g: (0, 0)),
        pl.BlockSpec((DIN, 512), lambda g: (0, g)),
        pl.BlockSpec((512,), lambda g: (g,)),
    ],
    out_specs=pl.BlockSpec((B * T, 512), lambda g: (0, g)),
    out_shape=jax.ShapeDtypeStruct((B * T, G), jnp.float32),
)


def _splat(v, j):
    idx = jnp.full((16, 1), j, dtype=jnp.int32)
    dnums = lax.GatherDimensionNumbers(
        offset_dims=(), collapsed_slice_dims=(0,), start_index_map=(0,)
    )
    return lax.gather(
        v, idx, dnums, slice_sizes=(1,),
        mode=lax.GatherScatterMode.PROMISE_IN_BOUNDS,
    )


def _sigmoid(x):
    return 1.0 / (1.0 + jnp.exp(-x))


def _tanh(x):
    return 2.0 / (1.0 + jnp.exp(-2.0 * x)) - 1.0


def _sc_body(xp_hbm, rc_hbm, vals_hbm, out_hbm,
             rc_v, vb0, vb1, vb2, rw0, rw1, rw2, cl0, cl1, cl2,
             gb0, gb1, gb2, ibuf, fbuf, g2buf, obuf, cbuf, hbuf,
             h_sh, gates_sh, semG, semS, semV):
    c = lax.axis_index("c")
    s = lax.axis_index("s")
    tb = s * NNZ_T
    n0 = s * NSEG
    g0 = s * GSEG

    vb = (vb0, vb1, vb2)
    rw = (rw0, rw1, rw2)
    cl = (cl0, cl1, cl2)
    gb = (gb0, gb1, gb2)

    pltpu.sync_copy(rc_hbm.at[pl.ds(tb, NNZ_T)], rc_v)

    def _unpack(k, p):
        def body(j, carry):
            for gg in range(8):
                v = rc_v[pl.ds(k * CH + j * 128 + gg * 16, 16)]
                rw[p][j, pl.ds(gg * 16, 16)] = jnp.right_shift(v, 14)
                cl[p][j, pl.ds(gg * 16, 16)] = jnp.bitwise_and(v, 0x3FFF)
            return carry
        lax.fori_loop(0, NSTR, body, 0)

    def _fire_gather(k, p):
        pltpu.async_copy(vals_hbm.at[pl.ds(tb + k * CH, CH)], vb[p], semV)
        for j in range(NSTR):
            pltpu.async_copy(
                h_sh.at[rw[p].at[j]], gb[p].at[pl.ds(j * 128, 128)], semG
            )

    def _wait_gather(p):
        pltpu.make_async_copy(vals_hbm.at[pl.ds(0, CH)], vb[p], semV).wait()
        for j in range(NSTR):
            pltpu.make_async_copy(
                h_sh.at[rw[p].at[j]], gb[p].at[pl.ds(j * 128, 128)], semG
            ).wait()

    def _fire_scatter(p):
        for j in range(NSTR):
            pltpu.async_copy(
                gb[p].at[pl.ds(j * 128, 128)],
                gates_sh.at[cl[p].at[j]],
                semS,
                add=True,
            )

    def _wait_scatter(p):
        for j in range(NSTR):
            pltpu.make_async_copy(
                gb[p].at[pl.ds(j * 128, 128)], gates_sh.at[cl[p].at[j]], semS
            ).wait()

    def _mult(p):
        def body(q, carry):
            vv = vb[p][pl.ds(q * 16, 16)]
            base = q * 16
            for j in range(16):
                gb[p][base + j] = gb[p][base + j] * _splat(vv, j)
            return carry
        lax.fori_loop(0, CH // 16, body, 0)

    def _zero(r, carry):
        hbuf[r] = jnp.zeros((HB,), jnp.float32)
        cbuf[r] = jnp.zeros((HB,), jnp.float32)
        return carry

    lax.fori_loop(0, NSEG, _zero, 0)
    pltpu.sync_copy(hbuf, h_sh.at[pl.ds(n0, NSEG)])
    plsc.subcore_barrier()

    def _step(t, carry):
        pltpu.sync_copy(xp_hbm.at[c, t, pl.ds(g0, GSEG)], gates_sh.at[pl.ds(g0, GSEG)])
        plsc.subcore_barrier()

        _unpack(0, 0)
        _fire_gather(0, 0)

        def _trip(i, carry2):
            for off in range(3):
                k = 3 * i + off
                p = off
                pn = (off + 1) % 3

                @pl.when(k >= 2)
                def _():
                    _wait_scatter(pn)

                @pl.when(k + 1 < NCHUNK)
                def _():
                    _unpack(k + 1, pn)
                    _fire_gather(k + 1, pn)

                _wait_gather(p)
                _mult(p)
                _fire_scatter(p)
            return carry2

        lax.fori_loop(0, NCHUNK // 3, _trip, 0)
        _wait_scatter((NCHUNK - 2) % 3)
        _wait_scatter((NCHUNK - 1) % 3)
        plsc.subcore_barrier()

        pltpu.sync_copy(gates_sh.at[pl.ds(n0, NSEG)], ibuf)
        pltpu.sync_copy(gates_sh.at[pl.ds(N + n0, NSEG)], fbuf)
        pltpu.sync_copy(gates_sh.at[pl.ds(2 * N + n0, NSEG)], g2buf)
        pltpu.sync_copy(gates_sh.at[pl.ds(3 * N + n0, NSEG)], obuf)

        def _ew(r, carry2):
            cn = _sigmoid(fbuf[r]) * cbuf[r] + _sigmoid(ibuf[r]) * _tanh(g2buf[r])
            cbuf[r] = cn
            hbuf[r] = _sigmoid(obuf[r]) * _tanh(cn)
            return carry2

        lax.fori_loop(0, NSEG, _ew, 0)
        pltpu.sync_copy(hbuf, h_sh.at[pl.ds(n0, NSEG)])
        pltpu.sync_copy(hbuf, out_hbm.at[c, t, pl.ds(n0, NSEG)])
        plsc.subcore_barrier()
        return carry

    lax.fori_loop(0, T, _step, 0)


_sc_call = pl.kernel(
    _sc_body,
    out_type=jax.ShapeDtypeStruct((NC, T, N, HB), jnp.float32),
    mesh=plsc.VectorSubcoreMesh(core_axis_name="c", subcore_axis_name="s"),
    scratch_types=[
        pltpu.VMEM((NNZ_T,), jnp.int32),
        pltpu.VMEM((CH,), jnp.float32),
        pltpu.VMEM((CH,), jnp.float32),
        pltpu.VMEM((CH,), jnp.float32),
        pltpu.VMEM((NSTR, 128), jnp.int32),
        pltpu.VMEM((NSTR, 128), jnp.int32),
        pltpu.VMEM((NSTR, 128), jnp.int32),
        pltpu.VMEM((NSTR, 128), jnp.int32),
        pltpu.VMEM((NSTR, 128), jnp.int32),
        pltpu.VMEM((NSTR, 128), jnp.int32),
        pltpu.VMEM((CH, HB), jnp.float32),
        pltpu.VMEM((CH, HB), jnp.float32),
        pltpu.VMEM((CH, HB), jnp.float32),
        pltpu.VMEM((NSEG, HB), jnp.float32),
        pltpu.VMEM((NSEG, HB), jnp.float32),
        pltpu.VMEM((NSEG, HB), jnp.float32),
        pltpu.VMEM((NSEG, HB), jnp.float32),
        pltpu.VMEM((NSEG, HB), jnp.float32),
        pltpu.VMEM((NSEG, HB), jnp.float32),
        pltpu.VMEM_SHARED((N, HB), jnp.float32),
        pltpu.VMEM_SHARED((G, HB), jnp.float32),
        pltpu.SemaphoreType.DMA,
        pltpu.SemaphoreType.DMA,
        pltpu.SemaphoreType.DMA,
    ],
    compiler_params=pltpu.CompilerParams(use_tc_tiling_on_sc=False),
)


def kernel(inputs, W_in, rows, cols, vals, bias):
    xproj = _xproj_call(inputs.reshape(B * T, DIN), W_in, bias)
    xp4 = xproj.reshape(NC, HB, T, G).transpose(0, 2, 3, 1)

    rc = rows * jnp.int32(G) + cols
    npad = NNZ_PAD - NNZ
    pad_ar = jnp.arange(npad, dtype=jnp.int32)
    rc_p = jnp.concatenate([rc, (pad_ar % N) * jnp.int32(G) + pad_ar % G])
    vals_p = jnp.concatenate([vals, jnp.zeros((npad,), jnp.float32)])

    hs4 = _sc_call(xp4, rc_p, vals_p)
    return hs4.transpose(0, 3, 1, 2).reshape(B, T, N)
